# Initial kernel scaffold; baseline (speedup 1.0000x reference)
import jax
import jax.numpy as jnp
from jax.experimental import pallas as pl


def kernel(x, weight, bias, running_mean, running_var):
    raise NotImplementedError("write your pallas kernel here")


# speedup vs baseline: 1.4529x; 1.4529x over previous
import functools

import jax
import jax.numpy as jnp
from jax.experimental import pallas as pl
from jax.experimental.pallas import tpu as pltpu


def _bn_fused_kernel(x_ref, w_ref, b_ref, o_ref, mean_ref, var_ref,
                     *, count, eps):
    x = x_ref[...]
    s = jnp.sum(x, axis=(0, 2))
    ss = jnp.sum(x * x, axis=(0, 2))
    inv = 1.0 / count
    mean = s * inv
    var = jnp.maximum(ss * inv - mean * mean, 0.0)
    scale = w_ref[:, 0] * jax.lax.rsqrt(var + eps)
    shift = b_ref[:, 0] - mean * scale
    o_ref[...] = x * scale[None, :, None] + shift[None, :, None]
    mean_ref[...] = mean[:, None]
    var_ref[...] = var[:, None]


def _pick_cb(n, c, hw, itemsize, vmem_budget=40 * 1024 * 1024):
    best = None
    cb = 8
    while cb <= c:
        if c % cb == 0:
            slab = n * cb * hw * itemsize
            if 4 * slab <= vmem_budget:
                best = cb
        cb += 8
    return best


def kernel(x, weight, bias, running_mean, running_var,
           *, eps=1e-5, momentum=0.1):
    N, C, H, W = x.shape
    HW = H * W
    x3 = x.reshape(N, C, HW)
    itemsize = jnp.dtype(x.dtype).itemsize
    CB = _pick_cb(N, C, HW, max(itemsize, 4))

    body = functools.partial(_bn_fused_kernel,
                             count=float(N * HW), eps=float(eps))
    y3, mean, var = pl.pallas_call(
        body,
        out_shape=(jax.ShapeDtypeStruct((N, C, HW), x.dtype),
                   jax.ShapeDtypeStruct((C, 1), jnp.float32),
                   jax.ShapeDtypeStruct((C, 1), jnp.float32)),
        grid=(C // CB,),
        in_specs=[pl.BlockSpec((N, CB, HW), lambda c: (0, c, 0)),
                  pl.BlockSpec((CB, 1), lambda c: (c, 0)),
                  pl.BlockSpec((CB, 1), lambda c: (c, 0))],
        out_specs=(pl.BlockSpec((N, CB, HW), lambda c: (0, c, 0)),
                   pl.BlockSpec((CB, 1), lambda c: (c, 0)),
                   pl.BlockSpec((CB, 1), lambda c: (c, 0))),
        compiler_params=pltpu.CompilerParams(
            dimension_semantics=("parallel",),
            vmem_limit_bytes=56 * 1024 * 1024),
    )(x3, weight.reshape(C, 1).astype(jnp.float32),
      bias.reshape(C, 1).astype(jnp.float32))

    y = y3.reshape(N, C, H, W)
    mean = mean[:, 0]
    var = var[:, 0]
    new_running_mean = running_mean + momentum * (
        mean.astype(running_mean.dtype) - running_mean)
    new_running_var = running_var + momentum * (
        var.astype(running_var.dtype) - running_var)
    return y, new_running_mean, new_running_var
